# baseline (speedup 1.0000x reference)
import functools

import jax
import jax.numpy as jnp
from jax import lax
from jax.experimental import pallas as pl
from jax.experimental.pallas import tpu as pltpu
from jax.experimental.pallas import tpu_sc as plsc

D_MODEL = 64
B_TOTAL = 4096 * 200
NUM_WORKERS = 32
B_PER_W = B_TOTAL // NUM_WORKERS
CHUNK = 128
N_CHUNKS = B_PER_W // CHUNK
SCALE = 8.0
LANES = 16


@functools.partial(
    pl.kernel,
    mesh=plsc.VectorSubcoreMesh(core_axis_name="c", subcore_axis_name="s"),
    out_type=jax.ShapeDtypeStruct((B_TOTAL, D_MODEL), jnp.float32),
    compiler_params=pltpu.CompilerParams(use_tc_tiling_on_sc=False),
    scratch_types=[
        pltpu.VMEM((N_CHUNKS, CHUNK), jnp.int32),
        pltpu.VMEM((CHUNK, D_MODEL), jnp.float32),
        pltpu.SemaphoreType.DMA,
    ],
)
def _emb_lookup(idx_hbm, table_hbm, out_hbm, idx_v, rows_v, sem):
    wid = lax.axis_index("s") * 2 + lax.axis_index("c")
    base = wid * B_PER_W
    pltpu.sync_copy(idx_hbm.at[pl.ds(wid * N_CHUNKS, N_CHUNKS)], idx_v)

    def chunk_body(g, carry):
        pltpu.async_copy(table_hbm.at[idx_v.at[g]], rows_v, sem).wait()

        def row_body(r, c):
            for j in range(D_MODEL // LANES):
                rows_v[r, pl.ds(j * LANES, LANES)] = (
                    rows_v[r, pl.ds(j * LANES, LANES)] * SCALE
                )
            return c

        lax.fori_loop(0, CHUNK, row_body, 0)
        pltpu.sync_copy(rows_v, out_hbm.at[pl.ds(base + g * CHUNK, CHUNK)])
        return carry

    lax.fori_loop(0, N_CHUNKS, chunk_body, 0)


def kernel(x, table):
    idx = x.reshape(-1).astype(jnp.int32).reshape(-1, CHUNK)
    out = _emb_lookup(idx, table)
    return out.reshape(x.shape + (D_MODEL,))


# speedup vs baseline: 1.2125x; 1.2125x over previous
import functools

import jax
import jax.numpy as jnp
from jax import lax
from jax.experimental import pallas as pl
from jax.experimental.pallas import tpu as pltpu
from jax.experimental.pallas import tpu_sc as plsc

D_MODEL = 64
B_TOTAL = 4096 * 200
NUM_WORKERS = 32
B_PER_W = B_TOTAL // NUM_WORKERS
CHUNK = 128
N_CHUNKS = B_PER_W // CHUNK
SCALE = 8.0
LANES = 16
NBUF = 4


@functools.partial(
    pl.kernel,
    mesh=plsc.VectorSubcoreMesh(core_axis_name="c", subcore_axis_name="s"),
    out_type=jax.ShapeDtypeStruct((B_TOTAL, D_MODEL), jnp.float32),
    compiler_params=pltpu.CompilerParams(use_tc_tiling_on_sc=False),
    scratch_types=[
        pltpu.VMEM((N_CHUNKS, CHUNK), jnp.int32),
        pltpu.VMEM((NBUF, CHUNK, D_MODEL), jnp.float32),
        pltpu.VMEM((NBUF, CHUNK, D_MODEL), jnp.float32),
        pltpu.SemaphoreType.DMA((NBUF,)),
        pltpu.SemaphoreType.DMA((NBUF,)),
    ],
)
def _emb_lookup(idx_hbm, table_hbm, out_hbm, idx_v, buf_g, buf_s, gsem, ssem):
    wid = lax.axis_index("s") * 2 + lax.axis_index("c")
    base = wid * B_PER_W
    pltpu.sync_copy(idx_hbm.at[pl.ds(wid * N_CHUNKS, N_CHUNKS)], idx_v)

    def start_gather(g, b):
        pltpu.async_copy(table_hbm.at[idx_v.at[g]], buf_g.at[b], gsem.at[b])

    for b in range(NBUF):
        start_gather(b, b)

    @pl.loop(0, N_CHUNKS, step=NBUF)
    def _outer(i0):
        for b in range(NBUF):
            i = i0 + b
            pltpu.make_async_copy(table_hbm.at[idx_v.at[i]],
                                  buf_g.at[b], gsem.at[b]).wait()
            @pl.when(i >= NBUF)
            def _():
                pltpu.make_async_copy(
                    buf_s.at[b],
                    out_hbm.at[pl.ds(base + (i - NBUF) * CHUNK, CHUNK)],
                    ssem.at[b]).wait()

            def row_body(r, c):
                for j in range(D_MODEL // LANES):
                    sl = pl.ds(j * LANES, LANES)
                    buf_s[b, r, sl] = buf_g[b, r, sl] * SCALE
                return c

            lax.fori_loop(0, CHUNK, row_body, 0)

            @pl.when(i + NBUF < N_CHUNKS)
            def _():
                start_gather(i + NBUF, b)

            pltpu.async_copy(
                buf_s.at[b],
                out_hbm.at[pl.ds(base + i * CHUNK, CHUNK)],
                ssem.at[b])

    for b in range(NBUF):
        i = N_CHUNKS - NBUF + b
        pltpu.make_async_copy(
            buf_s.at[b],
            out_hbm.at[pl.ds(base + i * CHUNK, CHUNK)],
            ssem.at[b]).wait()


def kernel(x, table):
    idx = x.reshape(-1).astype(jnp.int32).reshape(-1, CHUNK)
    out = _emb_lookup(idx, table)
    return out.reshape(x.shape + (D_MODEL,))
